# baseline (speedup 1.0000x reference)
import functools

import jax
import jax.numpy as jnp
from jax import lax
from jax.experimental import pallas as pl
from jax.experimental.pallas import tpu as pltpu
from jax.experimental.pallas import tpu_sc as plsc

BATCH = 16384
EMB = 64
H1 = 128
CHUNK = 128
LR_BIG = 12
LR_SMALL = 9


def _pack_pairs_body(a_ref, b_ref, c_ref, d_ref, o_ref):
    r = a_ref.shape[1]
    o_ref[0:r, 0:EMB] = jnp.swapaxes(a_ref[...], 0, 1)
    o_ref[0:r, EMB:2 * EMB] = jnp.swapaxes(b_ref[...], 0, 1)
    o_ref[r:2 * r, 0:EMB] = jnp.swapaxes(c_ref[...], 0, 1)
    o_ref[r:2 * r, EMB:2 * EMB] = jnp.swapaxes(d_ref[...], 0, 1)


def _pack_pairs(tT, lr):
    r = 1 << lr
    n = tT.shape[1]
    nblk = (n + 4 * r - 1) // (4 * r)
    last = (n - 1) // r

    def strip(k):
        return pl.BlockSpec((EMB, r), lambda i: (0, jnp.minimum(4 * i + k, last)))

    return pl.pallas_call(
        _pack_pairs_body,
        grid=(nblk,),
        in_specs=[strip(0), strip(1), strip(2), strip(3)],
        out_specs=pl.BlockSpec((2 * r, 2 * EMB), lambda i: (i, 0)),
        out_shape=jax.ShapeDtypeStruct((nblk * 2 * r, 2 * EMB), jnp.float32),
    )(tT, tT, tT, tT)


def _gather3_pairs(uid2d, did2d, bid2d, ut_p, dt_p, bt_p):
    info = plsc.get_sparse_core_info()
    nc, ns = info.num_cores, info.num_subcores
    nw = nc * ns
    rows_per_w = BATCH // nw
    nchunk = rows_per_w // CHUNK

    mesh = plsc.VectorSubcoreMesh(core_axis_name="c", subcore_axis_name="s")

    @functools.partial(
        pl.kernel,
        mesh=mesh,
        compiler_params=pltpu.CompilerParams(use_tc_tiling_on_sc=True),
        out_type=(
            jax.ShapeDtypeStruct((BATCH, 2 * EMB), jnp.float32),
            jax.ShapeDtypeStruct((BATCH, 2 * EMB), jnp.float32),
            jax.ShapeDtypeStruct((BATCH, 2 * EMB), jnp.float32),
        ),
        scratch_types=[
            pltpu.VMEM((nchunk, CHUNK), jnp.int32),
            pltpu.VMEM((nchunk, CHUNK), jnp.int32),
            pltpu.VMEM((nchunk, CHUNK), jnp.int32),
            pltpu.VMEM((rows_per_w, 2 * EMB), jnp.float32),
            pltpu.SemaphoreType.DMA,
        ],
    )
    def gather_kernel(uid_hbm, did_hbm, bid_hbm, ut_hbm, dt_hbm, bt_hbm,
                      uo_hbm, do_hbm, bo_hbm,
                      uidx_v, didx_v, bidx_v, rows_v, sem):
        wid = lax.axis_index("s") * nc + lax.axis_index("c")
        rbase = wid * nchunk
        pltpu.sync_copy(uid_hbm.at[pl.ds(rbase, nchunk)], uidx_v)
        pltpu.sync_copy(did_hbm.at[pl.ds(rbase, nchunk)], didx_v)
        pltpu.sync_copy(bid_hbm.at[pl.ds(rbase, nchunk)], bidx_v)
        for idx_v, lr in ((uidx_v, LR_BIG), (didx_v, LR_BIG), (bidx_v, LR_SMALL)):
            for c in range(nchunk):
                for k in range(CHUNK // 16):
                    s = pl.ds(k * 16, 16)
                    v = idx_v[c, s]
                    blk = lax.shift_right_logical(v, lr + 1)
                    off = lax.bitwise_and(v, (1 << lr) - 1)
                    idx_v[c, s] = lax.bitwise_or(
                        lax.shift_left(blk, lr), off)
        base = wid * rows_per_w
        for idx_v, t_hbm, o_hbm in ((uidx_v, ut_hbm, uo_hbm),
                                    (didx_v, dt_hbm, do_hbm),
                                    (bidx_v, bt_hbm, bo_hbm)):
            copies = [
                pltpu.async_copy(t_hbm.at[idx_v.at[c]],
                                 rows_v.at[pl.ds(c * CHUNK, CHUNK)], sem)
                for c in range(nchunk)
            ]
            for cp in copies:
                cp.wait()
            pltpu.sync_copy(rows_v, o_hbm.at[pl.ds(base, rows_per_w)])

    return gather_kernel(uid2d, did2d, bid2d, ut_p, dt_p, bt_p)


def _mlp_body(u_ref, d_ref, b_ref, um_ref, dm_ref, bm_ref,
              w1_ref, b1_ref, w2_ref, b2_ref, w3_ref, b3_ref, o_ref):
    def pick(pair_ref, m_ref):
        m = m_ref[...] > 0.5
        return jnp.where(m, pair_ref[:, EMB:2 * EMB], pair_ref[:, 0:EMB])

    u = pick(u_ref, um_ref)
    d = pick(d_ref, dm_ref)
    b = pick(b_ref, bm_ref)
    h = jnp.dot(u, w1_ref[0:EMB, :], preferred_element_type=jnp.float32)
    h = h + jnp.dot(d, w1_ref[EMB:2 * EMB, :], preferred_element_type=jnp.float32)
    h = h + jnp.dot(b, w1_ref[2 * EMB:3 * EMB, :], preferred_element_type=jnp.float32)
    h = jnp.maximum(h + b1_ref[...], 0.0)
    h = jnp.maximum(jnp.dot(h, w2_ref[...], preferred_element_type=jnp.float32) + b2_ref[...], 0.0)
    o = jnp.dot(h, w3_ref[...], preferred_element_type=jnp.float32) + b3_ref[...]
    o_ref[...] = o


def _mlp(u2, d2, b2_, um, dm, bm, W1, b1, W2, b2, W3, b3):
    bb = 2048
    grid = (BATCH // bb,)
    return pl.pallas_call(
        _mlp_body,
        grid=grid,
        in_specs=[
            pl.BlockSpec((bb, 2 * EMB), lambda i: (i, 0)),
            pl.BlockSpec((bb, 2 * EMB), lambda i: (i, 0)),
            pl.BlockSpec((bb, 2 * EMB), lambda i: (i, 0)),
            pl.BlockSpec((bb, 1), lambda i: (i, 0)),
            pl.BlockSpec((bb, 1), lambda i: (i, 0)),
            pl.BlockSpec((bb, 1), lambda i: (i, 0)),
            pl.BlockSpec((3 * EMB, H1), lambda i: (0, 0)),
            pl.BlockSpec((1, H1), lambda i: (0, 0)),
            pl.BlockSpec((H1, EMB), lambda i: (0, 0)),
            pl.BlockSpec((1, EMB), lambda i: (0, 0)),
            pl.BlockSpec((EMB, 1), lambda i: (0, 0)),
            pl.BlockSpec((1, 1), lambda i: (0, 0)),
        ],
        out_specs=pl.BlockSpec((bb, 1), lambda i: (i, 0)),
        out_shape=jax.ShapeDtypeStruct((BATCH, 1), jnp.float32),
    )(u2, d2, b2_, um, dm, bm, W1, b1.reshape(1, H1), W2, b2.reshape(1, EMB),
      W3, b3.reshape(1, 1))


def kernel(user_ids, device_ids, brand_ids, user_table, device_table, brand_table,
           W1, b1, W2, b2, W3, b3):
    uid = user_ids.astype(jnp.int32)
    did = device_ids.astype(jnp.int32)
    bid = brand_ids.astype(jnp.int32)
    ut_p = _pack_pairs(user_table.T, LR_BIG)
    dt_p = _pack_pairs(device_table.T, LR_BIG)
    bt_p = _pack_pairs(brand_table.T, LR_SMALL)
    u2, d2, b2_ = _gather3_pairs(
        uid.reshape(BATCH // CHUNK, CHUNK), did.reshape(BATCH // CHUNK, CHUNK),
        bid.reshape(BATCH // CHUNK, CHUNK), ut_p, dt_p, bt_p)
    um = ((uid >> LR_BIG) & 1).astype(jnp.float32).reshape(BATCH, 1)
    dm = ((did >> LR_BIG) & 1).astype(jnp.float32).reshape(BATCH, 1)
    bm = ((bid >> LR_SMALL) & 1).astype(jnp.float32).reshape(BATCH, 1)
    out = _mlp(u2, d2, b2_, um, dm, bm, W1, b1, W2, b2, W3, b3)
    return out.reshape(BATCH)


# speedup vs baseline: 1.0241x; 1.0241x over previous
import functools

import jax
import jax.numpy as jnp
from jax import lax
from jax.experimental import pallas as pl
from jax.experimental.pallas import tpu as pltpu
from jax.experimental.pallas import tpu_sc as plsc

BATCH = 16384
EMB = 64
H1 = 128
CHUNK = 128
LR_BIG = 12
LR_SMALL = 9


def _pack_pairs_body(a_ref, b_ref, c_ref, d_ref, o_ref):
    r = a_ref.shape[1]
    o_ref[0:r, 0:EMB] = jnp.swapaxes(a_ref[...], 0, 1)
    o_ref[0:r, EMB:2 * EMB] = jnp.swapaxes(b_ref[...], 0, 1)
    o_ref[r:2 * r, 0:EMB] = jnp.swapaxes(c_ref[...], 0, 1)
    o_ref[r:2 * r, EMB:2 * EMB] = jnp.swapaxes(d_ref[...], 0, 1)


def _pack_pairs(tT, lr):
    r = 1 << lr
    n = tT.shape[1]
    nblk = (n + 4 * r - 1) // (4 * r)
    last = (n - 1) // r

    def strip(k):
        return pl.BlockSpec((EMB, r), lambda i: (0, jnp.minimum(4 * i + k, last)))

    return pl.pallas_call(
        _pack_pairs_body,
        grid=(nblk,),
        in_specs=[strip(0), strip(1), strip(2), strip(3)],
        out_specs=pl.BlockSpec((2 * r, 2 * EMB), lambda i: (i, 0)),
        out_shape=jax.ShapeDtypeStruct((nblk * 2 * r, 2 * EMB), jnp.float32),
    )(tT, tT, tT, tT)


def _gather3_pairs(uid2d, did2d, bid2d, ut_p, dt_p, bt_p):
    info = plsc.get_sparse_core_info()
    nc, ns = info.num_cores, info.num_subcores
    nw = nc * ns
    rows_per_w = BATCH // nw
    nchunk = rows_per_w // CHUNK

    mesh = plsc.VectorSubcoreMesh(core_axis_name="c", subcore_axis_name="s")

    @functools.partial(
        pl.kernel,
        mesh=mesh,
        compiler_params=pltpu.CompilerParams(use_tc_tiling_on_sc=True),
        out_type=(
            jax.ShapeDtypeStruct((BATCH, 2 * EMB), jnp.float32),
            jax.ShapeDtypeStruct((BATCH, 2 * EMB), jnp.float32),
            jax.ShapeDtypeStruct((BATCH, 2 * EMB), jnp.float32),
        ),
        scratch_types=[
            pltpu.VMEM((nchunk, CHUNK), jnp.int32),
            pltpu.VMEM((nchunk, CHUNK), jnp.int32),
            pltpu.VMEM((nchunk, CHUNK), jnp.int32),
            pltpu.VMEM((rows_per_w, 2 * EMB), jnp.float32),
            pltpu.SemaphoreType.DMA,
        ],
    )
    def gather_kernel(uid_hbm, did_hbm, bid_hbm, ut_hbm, dt_hbm, bt_hbm,
                      uo_hbm, do_hbm, bo_hbm,
                      uidx_v, didx_v, bidx_v, rows_v, sem):
        wid = lax.axis_index("s") * nc + lax.axis_index("c")
        rbase = wid * nchunk
        pltpu.sync_copy(uid_hbm.at[pl.ds(rbase, nchunk)], uidx_v)
        pltpu.sync_copy(did_hbm.at[pl.ds(rbase, nchunk)], didx_v)
        pltpu.sync_copy(bid_hbm.at[pl.ds(rbase, nchunk)], bidx_v)
        for idx_v, lr in ((uidx_v, LR_BIG), (didx_v, LR_BIG), (bidx_v, LR_SMALL)):
            for c in range(nchunk):
                for k in range(CHUNK // 16):
                    s = pl.ds(k * 16, 16)
                    v = idx_v[c, s]
                    blk = lax.shift_right_logical(v, lr + 1)
                    off = lax.bitwise_and(v, (1 << lr) - 1)
                    idx_v[c, s] = lax.bitwise_or(
                        lax.shift_left(blk, lr), off)
        base = wid * rows_per_w
        for idx_v, t_hbm, o_hbm in ((uidx_v, ut_hbm, uo_hbm),
                                    (didx_v, dt_hbm, do_hbm),
                                    (bidx_v, bt_hbm, bo_hbm)):
            copies = [
                pltpu.async_copy(t_hbm.at[idx_v.at[c]],
                                 rows_v.at[pl.ds(c * CHUNK, CHUNK)], sem)
                for c in range(nchunk)
            ]
            for cp in copies:
                cp.wait()
            pltpu.sync_copy(rows_v, o_hbm.at[pl.ds(base, rows_per_w)])

    return gather_kernel(uid2d, did2d, bid2d, ut_p, dt_p, bt_p)


def _mlp_body(u_ref, d_ref, b_ref, hm_ref,
              w1_ref, b1_ref, w2_ref, b2_ref, w3_ref, b3_ref, o_ref):
    hm = hm_ref[...]

    def pick(pair_ref, bit):
        m = (hm & bit) > 0
        return jnp.where(m, pair_ref[:, EMB:2 * EMB], pair_ref[:, 0:EMB])

    u = pick(u_ref, 1)
    d = pick(d_ref, 2)
    b = pick(b_ref, 4)
    h = jnp.dot(u, w1_ref[0:EMB, :], preferred_element_type=jnp.float32)
    h = h + jnp.dot(d, w1_ref[EMB:2 * EMB, :], preferred_element_type=jnp.float32)
    h = h + jnp.dot(b, w1_ref[2 * EMB:3 * EMB, :], preferred_element_type=jnp.float32)
    h = jnp.maximum(h + b1_ref[...], 0.0)
    h = jnp.maximum(jnp.dot(h, w2_ref[...], preferred_element_type=jnp.float32) + b2_ref[...], 0.0)
    o = jnp.dot(h, w3_ref[...], preferred_element_type=jnp.float32) + b3_ref[...]
    o_ref[...] = o


def _mlp(u2, d2, b2_, hm, W1, b1, W2, b2, W3, b3):
    bb = 2048
    grid = (BATCH // bb,)
    return pl.pallas_call(
        _mlp_body,
        grid=grid,
        in_specs=[
            pl.BlockSpec((bb, 2 * EMB), lambda i: (i, 0)),
            pl.BlockSpec((bb, 2 * EMB), lambda i: (i, 0)),
            pl.BlockSpec((bb, 2 * EMB), lambda i: (i, 0)),
            pl.BlockSpec((bb, 1), lambda i: (i, 0)),
            pl.BlockSpec((3 * EMB, H1), lambda i: (0, 0)),
            pl.BlockSpec((1, H1), lambda i: (0, 0)),
            pl.BlockSpec((H1, EMB), lambda i: (0, 0)),
            pl.BlockSpec((1, EMB), lambda i: (0, 0)),
            pl.BlockSpec((EMB, 1), lambda i: (0, 0)),
            pl.BlockSpec((1, 1), lambda i: (0, 0)),
        ],
        out_specs=pl.BlockSpec((bb, 1), lambda i: (i, 0)),
        out_shape=jax.ShapeDtypeStruct((BATCH, 1), jnp.float32),
    )(u2, d2, b2_, hm, W1, b1.reshape(1, H1), W2, b2.reshape(1, EMB),
      W3, b3.reshape(1, 1))


def kernel(user_ids, device_ids, brand_ids, user_table, device_table, brand_table,
           W1, b1, W2, b2, W3, b3):
    uid = user_ids.astype(jnp.int32)
    did = device_ids.astype(jnp.int32)
    bid = brand_ids.astype(jnp.int32)
    ut_p = _pack_pairs(user_table.T, LR_BIG)
    dt_p = _pack_pairs(device_table.T, LR_BIG)
    bt_p = _pack_pairs(brand_table.T, LR_SMALL)
    u2, d2, b2_ = _gather3_pairs(
        uid.reshape(BATCH // CHUNK, CHUNK), did.reshape(BATCH // CHUNK, CHUNK),
        bid.reshape(BATCH // CHUNK, CHUNK), ut_p, dt_p, bt_p)
    hm = (((uid >> LR_BIG) & 1) | (((did >> LR_BIG) & 1) << 1)
          | (((bid >> LR_SMALL) & 1) << 2)).reshape(BATCH, 1)
    out = _mlp(u2, d2, b2_, hm, W1, b1, W2, b2, W3, b3)
    return out.reshape(BATCH)


# speedup vs baseline: 1.0989x; 1.0731x over previous
import functools

import jax
import jax.numpy as jnp
from jax import lax
from jax.experimental import pallas as pl
from jax.experimental.pallas import tpu as pltpu
from jax.experimental.pallas import tpu_sc as plsc

BATCH = 16384
EMB = 64
H1 = 128
CHUNK = 128
LR_BIG = 11
LR_SMALL = 9


def _to_bf16_bits(x_i32):
    rnd = x_i32 + 0x7FFF + (lax.shift_right_logical(x_i32, 16) & 1)
    return rnd & jnp.int32(-65536)


def _pack_body(a_ref, b_ref, c_ref, d_ref, e_ref, f_ref, g_ref, h_ref, o_ref):
    r = a_ref.shape[1]
    eye = (lax.broadcasted_iota(jnp.int32, (EMB, EMB), 0)
           == lax.broadcasted_iota(jnp.int32, (EMB, EMB), 1)).astype(jnp.float32)

    def tp(ref):
        return lax.dot_general(ref[...], eye, (((0,), (0,)), ((), ())),
                               preferred_element_type=jnp.float32)

    def pack2(hi, lo):
        hb = _to_bf16_bits(lax.bitcast_convert_type(hi, jnp.int32))
        lb = _to_bf16_bits(lax.bitcast_convert_type(lo, jnp.int32))
        word = hb | lax.shift_right_logical(lb, 16)
        return lax.bitcast_convert_type(word, jnp.float32)

    o_ref[0:r, 0:EMB] = pack2(tp(a_ref), tp(b_ref))
    o_ref[0:r, EMB:2 * EMB] = pack2(tp(c_ref), tp(d_ref))
    o_ref[r:2 * r, 0:EMB] = pack2(tp(e_ref), tp(f_ref))
    o_ref[r:2 * r, EMB:2 * EMB] = pack2(tp(g_ref), tp(h_ref))


def _pack_quads(tT, lr):
    r = 1 << lr
    n = tT.shape[1]
    nblk = (n + 8 * r - 1) // (8 * r)
    last = (n - 1) // r

    def strip(k):
        return pl.BlockSpec((EMB, r), lambda i: (0, jnp.minimum(8 * i + k, last)))

    return pl.pallas_call(
        _pack_body,
        grid=(nblk,),
        in_specs=[strip(k) for k in range(8)],
        out_specs=pl.BlockSpec((2 * r, 2 * EMB), lambda i: (i, 0)),
        out_shape=jax.ShapeDtypeStruct((nblk * 2 * r, 2 * EMB), jnp.float32),
    )(*([tT] * 8))


def _gather3(uid2d, did2d, bid2d, ut_p, dt_p, bt_p):
    info = plsc.get_sparse_core_info()
    nc, ns = info.num_cores, info.num_subcores
    nw = nc * ns
    rows_per_w = BATCH // nw
    nchunk = rows_per_w // CHUNK

    mesh = plsc.VectorSubcoreMesh(core_axis_name="c", subcore_axis_name="s")

    @functools.partial(
        pl.kernel,
        mesh=mesh,
        compiler_params=pltpu.CompilerParams(use_tc_tiling_on_sc=True),
        out_type=(
            jax.ShapeDtypeStruct((BATCH, 2 * EMB), jnp.float32),
            jax.ShapeDtypeStruct((BATCH, 2 * EMB), jnp.float32),
            jax.ShapeDtypeStruct((BATCH, 2 * EMB), jnp.float32),
        ),
        scratch_types=[
            pltpu.VMEM((nchunk, CHUNK), jnp.int32),
            pltpu.VMEM((nchunk, CHUNK), jnp.int32),
            pltpu.VMEM((nchunk, CHUNK), jnp.int32),
            pltpu.VMEM((rows_per_w, 2 * EMB), jnp.float32),
            pltpu.SemaphoreType.DMA,
        ],
    )
    def gather_kernel(uid_hbm, did_hbm, bid_hbm, ut_hbm, dt_hbm, bt_hbm,
                      uo_hbm, do_hbm, bo_hbm,
                      uidx_v, didx_v, bidx_v, rows_v, sem):
        wid = lax.axis_index("s") * nc + lax.axis_index("c")
        rbase = wid * nchunk
        pltpu.sync_copy(uid_hbm.at[pl.ds(rbase, nchunk)], uidx_v)
        pltpu.sync_copy(did_hbm.at[pl.ds(rbase, nchunk)], didx_v)
        pltpu.sync_copy(bid_hbm.at[pl.ds(rbase, nchunk)], bidx_v)
        for idx_v, lr in ((uidx_v, LR_BIG), (didx_v, LR_BIG), (bidx_v, LR_SMALL)):
            for c in range(nchunk):
                for k in range(CHUNK // 16):
                    s = pl.ds(k * 16, 16)
                    v = idx_v[c, s]
                    blk = lax.shift_right_logical(v, lr + 2)
                    off = lax.bitwise_and(v, (1 << lr) - 1)
                    idx_v[c, s] = lax.bitwise_or(lax.shift_left(blk, lr), off)
        base = wid * rows_per_w
        for idx_v, t_hbm, o_hbm in ((uidx_v, ut_hbm, uo_hbm),
                                    (didx_v, dt_hbm, do_hbm),
                                    (bidx_v, bt_hbm, bo_hbm)):
            copies = [
                pltpu.async_copy(t_hbm.at[idx_v.at[c]],
                                 rows_v.at[pl.ds(c * CHUNK, CHUNK)], sem)
                for c in range(nchunk)
            ]
            for cp in copies:
                cp.wait()
            pltpu.sync_copy(rows_v, o_hbm.at[pl.ds(base, rows_per_w)])

    return gather_kernel(uid2d, did2d, bid2d, ut_p, dt_p, bt_p)


def _mlp_body(u_ref, d_ref, b_ref, hm_ref,
              w1_ref, b1_ref, w2_ref, b2_ref, w3_ref, b3_ref, o_ref):
    hm = hm_ref[...]

    def pick(pair_ref, h1bit, h0bit):
        h1 = (hm & h1bit) > 0
        h0 = (hm & h0bit) > 0
        half = jnp.where(h1, pair_ref[:, EMB:2 * EMB], pair_ref[:, 0:EMB])
        bits = lax.bitcast_convert_type(half, jnp.int32)
        bits = jnp.where(h0, lax.shift_left(bits, 16),
                         bits & jnp.int32(-65536))
        return lax.bitcast_convert_type(bits, jnp.float32)

    u = pick(u_ref, 1, 2)
    d = pick(d_ref, 4, 8)
    b = pick(b_ref, 16, 32)
    h = jnp.dot(u, w1_ref[0:EMB, :], preferred_element_type=jnp.float32)
    h = h + jnp.dot(d, w1_ref[EMB:2 * EMB, :], preferred_element_type=jnp.float32)
    h = h + jnp.dot(b, w1_ref[2 * EMB:3 * EMB, :], preferred_element_type=jnp.float32)
    h = jnp.maximum(h + b1_ref[...], 0.0)
    h = jnp.maximum(jnp.dot(h, w2_ref[...], preferred_element_type=jnp.float32) + b2_ref[...], 0.0)
    o = jnp.dot(h, w3_ref[...], preferred_element_type=jnp.float32) + b3_ref[...]
    o_ref[...] = o


def _mlp(u2, d2, b2_, hm, W1, b1, W2, b2, W3, b3):
    bb = 2048
    grid = (BATCH // bb,)
    return pl.pallas_call(
        _mlp_body,
        grid=grid,
        in_specs=[
            pl.BlockSpec((bb, 2 * EMB), lambda i: (i, 0)),
            pl.BlockSpec((bb, 2 * EMB), lambda i: (i, 0)),
            pl.BlockSpec((bb, 2 * EMB), lambda i: (i, 0)),
            pl.BlockSpec((bb, 1), lambda i: (i, 0)),
            pl.BlockSpec((3 * EMB, H1), lambda i: (0, 0)),
            pl.BlockSpec((1, H1), lambda i: (0, 0)),
            pl.BlockSpec((H1, EMB), lambda i: (0, 0)),
            pl.BlockSpec((1, EMB), lambda i: (0, 0)),
            pl.BlockSpec((EMB, 1), lambda i: (0, 0)),
            pl.BlockSpec((1, 1), lambda i: (0, 0)),
        ],
        out_specs=pl.BlockSpec((bb, 1), lambda i: (i, 0)),
        out_shape=jax.ShapeDtypeStruct((BATCH, 1), jnp.float32),
    )(u2, d2, b2_, hm, W1, b1.reshape(1, H1), W2, b2.reshape(1, EMB),
      W3, b3.reshape(1, 1))


def kernel(user_ids, device_ids, brand_ids, user_table, device_table, brand_table,
           W1, b1, W2, b2, W3, b3):
    uid = user_ids.astype(jnp.int32)
    did = device_ids.astype(jnp.int32)
    bid = brand_ids.astype(jnp.int32)
    ut_p = _pack_quads(user_table.T, LR_BIG)
    dt_p = _pack_quads(device_table.T, LR_BIG)
    bt_p = _pack_quads(brand_table.T, LR_SMALL)
    u2, d2, b2_ = _gather3(
        uid.reshape(BATCH // CHUNK, CHUNK), did.reshape(BATCH // CHUNK, CHUNK),
        bid.reshape(BATCH // CHUNK, CHUNK), ut_p, dt_p, bt_p)
    hm = ((((uid >> (LR_BIG + 1)) & 1) | (((uid >> LR_BIG) & 1) << 1))
          | ((((did >> (LR_BIG + 1)) & 1) << 2) | (((did >> LR_BIG) & 1) << 3))
          | ((((bid >> (LR_SMALL + 1)) & 1) << 4) | (((bid >> LR_SMALL) & 1) << 5))
          ).reshape(BATCH, 1)
    out = _mlp(u2, d2, b2_, hm, W1, b1, W2, b2, W3, b3)
    return out.reshape(BATCH)


# speedup vs baseline: 1.1281x; 1.0266x over previous
import functools

import jax
import jax.numpy as jnp
from jax import lax
from jax.experimental import pallas as pl
from jax.experimental.pallas import tpu as pltpu
from jax.experimental.pallas import tpu_sc as plsc

BATCH = 16384
EMB = 64
H1 = 128
CHUNK = 128
LR_BIG = 11
LR_SMALL = 9


def _to_bf16_bits(x_i32):
    return x_i32 & jnp.int32(-65536)


def _pack_body(a_ref, b_ref, c_ref, d_ref, e_ref, f_ref, g_ref, h_ref, o_ref):
    r = a_ref.shape[1]
    eye = (lax.broadcasted_iota(jnp.int32, (EMB, EMB), 0)
           == lax.broadcasted_iota(jnp.int32, (EMB, EMB), 1)).astype(jnp.float32)

    def tp(ref):
        return lax.dot_general(ref[...], eye, (((0,), (0,)), ((), ())),
                               preferred_element_type=jnp.float32)

    def pack2(hi, lo):
        hb = _to_bf16_bits(lax.bitcast_convert_type(hi, jnp.int32))
        lb = _to_bf16_bits(lax.bitcast_convert_type(lo, jnp.int32))
        word = hb | lax.shift_right_logical(lb, 16)
        return lax.bitcast_convert_type(word, jnp.float32)

    o_ref[0:r, 0:EMB] = pack2(tp(a_ref), tp(b_ref))
    o_ref[0:r, EMB:2 * EMB] = pack2(tp(c_ref), tp(d_ref))
    o_ref[r:2 * r, 0:EMB] = pack2(tp(e_ref), tp(f_ref))
    o_ref[r:2 * r, EMB:2 * EMB] = pack2(tp(g_ref), tp(h_ref))


def _pack_quads(tT, lr):
    r = 1 << lr
    n = tT.shape[1]
    nblk = (n + 8 * r - 1) // (8 * r)
    last = (n - 1) // r

    def strip(k):
        return pl.BlockSpec((EMB, r), lambda i: (0, jnp.minimum(8 * i + k, last)))

    return pl.pallas_call(
        _pack_body,
        grid=(nblk,),
        in_specs=[strip(k) for k in range(8)],
        out_specs=pl.BlockSpec((2 * r, 2 * EMB), lambda i: (i, 0)),
        out_shape=jax.ShapeDtypeStruct((nblk * 2 * r, 2 * EMB), jnp.float32),
    )(*([tT] * 8))


def _gather3(uid2d, did2d, bid2d, ut_p, dt_p, bt_p):
    info = plsc.get_sparse_core_info()
    nc, ns = info.num_cores, info.num_subcores
    nw = nc * ns
    rows_per_w = BATCH // nw
    nchunk = rows_per_w // CHUNK

    mesh = plsc.VectorSubcoreMesh(core_axis_name="c", subcore_axis_name="s")

    @functools.partial(
        pl.kernel,
        mesh=mesh,
        compiler_params=pltpu.CompilerParams(use_tc_tiling_on_sc=True),
        out_type=(
            jax.ShapeDtypeStruct((BATCH, 2 * EMB), jnp.float32),
            jax.ShapeDtypeStruct((BATCH, 2 * EMB), jnp.float32),
            jax.ShapeDtypeStruct((BATCH, 2 * EMB), jnp.float32),
        ),
        scratch_types=[
            pltpu.VMEM((nchunk, CHUNK), jnp.int32),
            pltpu.VMEM((nchunk, CHUNK), jnp.int32),
            pltpu.VMEM((nchunk, CHUNK), jnp.int32),
            pltpu.VMEM((rows_per_w, 2 * EMB), jnp.float32),
            pltpu.SemaphoreType.DMA,
        ],
    )
    def gather_kernel(uid_hbm, did_hbm, bid_hbm, ut_hbm, dt_hbm, bt_hbm,
                      uo_hbm, do_hbm, bo_hbm,
                      uidx_v, didx_v, bidx_v, rows_v, sem):
        wid = lax.axis_index("s") * nc + lax.axis_index("c")
        rbase = wid * nchunk
        pltpu.sync_copy(uid_hbm.at[pl.ds(rbase, nchunk)], uidx_v)
        pltpu.sync_copy(did_hbm.at[pl.ds(rbase, nchunk)], didx_v)
        pltpu.sync_copy(bid_hbm.at[pl.ds(rbase, nchunk)], bidx_v)
        for idx_v, lr in ((uidx_v, LR_BIG), (didx_v, LR_BIG), (bidx_v, LR_SMALL)):
            for c in range(nchunk):
                for k in range(CHUNK // 16):
                    s = pl.ds(k * 16, 16)
                    v = idx_v[c, s]
                    blk = lax.shift_right_logical(v, lr + 2)
                    off = lax.bitwise_and(v, (1 << lr) - 1)
                    idx_v[c, s] = lax.bitwise_or(lax.shift_left(blk, lr), off)
        base = wid * rows_per_w
        for idx_v, t_hbm, o_hbm in ((uidx_v, ut_hbm, uo_hbm),
                                    (didx_v, dt_hbm, do_hbm),
                                    (bidx_v, bt_hbm, bo_hbm)):
            copies = [
                pltpu.async_copy(t_hbm.at[idx_v.at[c]],
                                 rows_v.at[pl.ds(c * CHUNK, CHUNK)], sem)
                for c in range(nchunk)
            ]
            for cp in copies:
                cp.wait()
            pltpu.sync_copy(rows_v, o_hbm.at[pl.ds(base, rows_per_w)])

    return gather_kernel(uid2d, did2d, bid2d, ut_p, dt_p, bt_p)


def _mlp_body(u_ref, d_ref, b_ref, hm_ref,
              w1_ref, b1_ref, w2_ref, b2_ref, w3_ref, b3_ref, o_ref):
    hm = hm_ref[...]

    def pick(pair_ref, h1bit, h0bit):
        h1 = (hm & h1bit) > 0
        h0 = (hm & h0bit) > 0
        half = jnp.where(h1, pair_ref[:, EMB:2 * EMB], pair_ref[:, 0:EMB])
        bits = lax.bitcast_convert_type(half, jnp.int32)
        bits = jnp.where(h0, lax.shift_left(bits, 16),
                         bits & jnp.int32(-65536))
        return lax.bitcast_convert_type(bits, jnp.float32)

    u = pick(u_ref, 1, 2)
    d = pick(d_ref, 4, 8)
    b = pick(b_ref, 16, 32)
    h = jnp.dot(u, w1_ref[0:EMB, :], preferred_element_type=jnp.float32)
    h = h + jnp.dot(d, w1_ref[EMB:2 * EMB, :], preferred_element_type=jnp.float32)
    h = h + jnp.dot(b, w1_ref[2 * EMB:3 * EMB, :], preferred_element_type=jnp.float32)
    h = jnp.maximum(h + b1_ref[...], 0.0)
    h = jnp.maximum(jnp.dot(h, w2_ref[...], preferred_element_type=jnp.float32) + b2_ref[...], 0.0)
    o = jnp.dot(h, w3_ref[...], preferred_element_type=jnp.float32) + b3_ref[...]
    o_ref[...] = o


def _mlp(u2, d2, b2_, hm, W1, b1, W2, b2, W3, b3):
    bb = 2048
    grid = (BATCH // bb,)
    return pl.pallas_call(
        _mlp_body,
        grid=grid,
        in_specs=[
            pl.BlockSpec((bb, 2 * EMB), lambda i: (i, 0)),
            pl.BlockSpec((bb, 2 * EMB), lambda i: (i, 0)),
            pl.BlockSpec((bb, 2 * EMB), lambda i: (i, 0)),
            pl.BlockSpec((bb, 1), lambda i: (i, 0)),
            pl.BlockSpec((3 * EMB, H1), lambda i: (0, 0)),
            pl.BlockSpec((1, H1), lambda i: (0, 0)),
            pl.BlockSpec((H1, EMB), lambda i: (0, 0)),
            pl.BlockSpec((1, EMB), lambda i: (0, 0)),
            pl.BlockSpec((EMB, 1), lambda i: (0, 0)),
            pl.BlockSpec((1, 1), lambda i: (0, 0)),
        ],
        out_specs=pl.BlockSpec((bb, 1), lambda i: (i, 0)),
        out_shape=jax.ShapeDtypeStruct((BATCH, 1), jnp.float32),
    )(u2, d2, b2_, hm, W1, b1.reshape(1, H1), W2, b2.reshape(1, EMB),
      W3, b3.reshape(1, 1))


def kernel(user_ids, device_ids, brand_ids, user_table, device_table, brand_table,
           W1, b1, W2, b2, W3, b3):
    uid = user_ids.astype(jnp.int32)
    did = device_ids.astype(jnp.int32)
    bid = brand_ids.astype(jnp.int32)
    ut_p = _pack_quads(user_table.T, LR_BIG)
    dt_p = _pack_quads(device_table.T, LR_BIG)
    bt_p = _pack_quads(brand_table.T, LR_SMALL)
    u2, d2, b2_ = _gather3(
        uid.reshape(BATCH // CHUNK, CHUNK), did.reshape(BATCH // CHUNK, CHUNK),
        bid.reshape(BATCH // CHUNK, CHUNK), ut_p, dt_p, bt_p)
    hm = ((((uid >> (LR_BIG + 1)) & 1) | (((uid >> LR_BIG) & 1) << 1))
          | ((((did >> (LR_BIG + 1)) & 1) << 2) | (((did >> LR_BIG) & 1) << 3))
          | ((((bid >> (LR_SMALL + 1)) & 1) << 4) | (((bid >> LR_SMALL) & 1) << 5))
          ).reshape(BATCH, 1)
    out = _mlp(u2, d2, b2_, hm, W1, b1, W2, b2, W3, b3)
    return out.reshape(BATCH)
